# baseline (speedup 1.0000x reference)
import functools

import jax
import jax.numpy as jnp
from jax.experimental import pallas as pl
from jax.experimental.pallas import tpu as pltpu
from jax.experimental.pallas import tpu_sc as plsc

K = 3
G = 64
T = 1024
RB = 2000
EPS = 1e-5
BIG = 1e20
JINF = float("inf")
IBIG = 1 << 30

_PREC = jax.lax.Precision.DEFAULT


def _mmdot(a, b, prec=None):
    return jax.lax.dot_general(
        a, b, (((a.ndim - 1,), (0,)), ((), ())),
        precision=_PREC if prec is None else prec,
        preferred_element_type=jnp.float32)


def _less(a, b):
    return (a[0] < b[0]) | ((a[0] == b[0]) & (a[1] < b[1]))


def _sel(t, a, b):
    return jnp.where(t, a[0], b[0]), jnp.where(t, a[1], b[1])


def _lexmin(a, b):
    return _sel(_less(a, b), a, b)


def _lexmax(a, b):
    return _sel(_less(b, a), a, b)


def _merge3(A, B):
    a0, a1, a2 = A
    b0, b1, b2 = B
    t0 = _less(b0, a0)
    m0 = _sel(t0, b0, a0)
    l0 = _sel(t0, a0, b0)
    opp = _sel(t0, b1, a1)
    m1 = _lexmin(l0, opp)
    c1 = _lexmax(a0, b1)
    c2 = _lexmax(a1, b0)
    m2 = _lexmin(_lexmin(a2, b2), _lexmin(c1, c2))
    return m0, m1, m2


def _knn_body(lo_ref, hi_ref, x_ref, xT_ref, brow_ref, bcol_ref,
              i0_ref, i1_ref, i2_ref, *, n, feat):
    i = pl.program_id(0)
    r0 = i * T
    xr = x_ref[pl.ds(r0, T), 0:feat]
    sqr = jnp.sum(xr * xr, axis=1, keepdims=True)
    bcr = bcol_ref[pl.ds(r0, T), :]
    rowid = r0 + jax.lax.broadcasted_iota(jnp.int32, (T, 1), 0)
    lo = lo_ref[i]
    hi = hi_ref[i]

    def col_tile(j, carry):
        v0, i0, v1, i1, v2, i2 = carry
        c0 = j * T
        xc = xT_ref[:, pl.ds(c0, T)]
        sqc = jnp.sum(xc * xc, axis=0, keepdims=True)
        dot = _mmdot(xr, xc)
        d2 = sqr + sqc - 2.0 * dot
        bcc = brow_ref[:, pl.ds(c0, T)]
        colid = c0 + jax.lax.broadcasted_iota(jnp.int32, (T, T), 1)
        d2 = jnp.where((bcr != bcc) | (rowid == colid), BIG, d2)
        d2 = jnp.where(colid >= n, JINF, d2)
        tile = []
        d = d2
        for s in range(3):
            mv = jnp.min(d, axis=1, keepdims=True)
            mi = jnp.min(jnp.where(d == mv, colid, IBIG), axis=1, keepdims=True)
            tile.append((mv, mi))
            if s < 2:
                d = jnp.where(colid == mi, JINF, d)
        (v0, i0), (v1, i1), (v2, i2) = _merge3(
            ((v0, i0), (v1, i1), (v2, i2)), tuple(tile))
        return v0, i0, v1, i1, v2, i2

    finf = jnp.full((T, 1), JINF, jnp.float32)
    init = (finf, jnp.full((T, 1), IBIG, jnp.int32),
            finf, jnp.full((T, 1), IBIG + 1, jnp.int32),
            finf, jnp.full((T, 1), IBIG + 2, jnp.int32))
    v0, i0, v1, i1, v2, i2 = jax.lax.fori_loop(lo, hi + 1, col_tile, init)

    hc = (hi + 1) * T
    base = jnp.where(lo > 0, 0, hc)
    ones_f = jnp.ones((T, 1), jnp.float32)
    ones_i = jnp.ones((T, 1), jnp.int32)
    virt = []
    for s in range(3):
        vidx = base + s
        vval = jnp.where(vidx < n, BIG, JINF)
        virt.append((vval * ones_f, vidx * ones_i))
    (v0, i0), (v1, i1), (v2, i2) = _merge3(
        ((v0, i0), (v1, i1), (v2, i2)), tuple(virt))

    i0_ref[...] = i0
    i1_ref[...] = i1
    i2_ref[...] = i2


def _knn(xp, xT, brow, bcol, lo_t, hi_t, n, feat):
    npad = xp.shape[0]
    nt = npad // T
    body = functools.partial(_knn_body, n=n, feat=feat)
    grid_spec = pltpu.PrefetchScalarGridSpec(
        num_scalar_prefetch=2,
        grid=(nt,),
        in_specs=[
            pl.BlockSpec(xp.shape, lambda i, *_: (0, 0)),
            pl.BlockSpec(xT.shape, lambda i, *_: (0, 0)),
            pl.BlockSpec(brow.shape, lambda i, *_: (0, 0)),
            pl.BlockSpec(bcol.shape, lambda i, *_: (0, 0)),
        ],
        out_specs=[
            pl.BlockSpec((T, 1), lambda i, *_: (i, 0)),
            pl.BlockSpec((T, 1), lambda i, *_: (i, 0)),
            pl.BlockSpec((T, 1), lambda i, *_: (i, 0)),
        ],
    )
    out_shape = [jax.ShapeDtypeStruct((npad, 1), jnp.int32)] * 3
    return pl.pallas_call(
        body, grid_spec=grid_spec, out_shape=out_shape,
        compiler_params=pltpu.CompilerParams(
            dimension_semantics=("parallel",)),
    )(lo_t, hi_t, xp, xT, brow, bcol)


def _sc_gather(table, idx, width, chunks):
    nc, ns = 2, 16
    nw = nc * ns
    ep = idx.shape[0]
    bpw = ep // nw
    cs = bpw // chunks
    mesh = plsc.VectorSubcoreMesh(core_axis_name="c", subcore_axis_name="s")

    @functools.partial(
        pl.kernel, mesh=mesh,
        out_type=jax.ShapeDtypeStruct((ep, width), jnp.float32),
        scratch_types=[
            pltpu.VMEM((bpw,), jnp.int32),
            pltpu.VMEM((cs, width), jnp.float32),
            pltpu.SemaphoreType.DMA,
        ])
    def k(table_hbm, idx_hbm, out_hbm, idx_v, rows_v, sem):
        wid = jax.lax.axis_index("s") * nc + jax.lax.axis_index("c")
        base = wid * bpw
        pltpu.sync_copy(idx_hbm.at[pl.ds(base, bpw)], idx_v)
        for c in range(chunks):
            src = idx_v if chunks == 1 else idx_v.at[pl.ds(c * cs, cs)]
            pltpu.async_copy(table_hbm.at[src], rows_v, sem).wait()
            pltpu.sync_copy(rows_v, out_hbm.at[pl.ds(base + c * cs, cs)])

    return k(table, idx)


def _ec_common(x_ref, g_ref, w_ref, b_ref, ga_ref, be_ref, msg_ref, xn_ref,
               e_s, s1_s, s2_s, *, ne, do, de, nb):
    p = pl.program_id(0)
    b = pl.program_id(1)
    rb = x_ref.shape[0]
    e_s[:, 0:de] = x_ref[:, 0:de]

    def h_of(t):
        e_s[:, de:2 * de] = g_ref[pl.Slice(t, rb, 3), 0:de]
        return _mmdot(e_s[...], w_ref[...]) + b_ref[...]

    @pl.when(p == 0)
    def _():
        @pl.when(b == 0)
        def _():
            s1_s[...] = jnp.zeros_like(s1_s)
            s2_s[...] = jnp.zeros_like(s2_s)
        for t in range(3):
            h = h_of(t)
            s1_s[...] += jnp.sum(h, axis=0, keepdims=True)
            s2_s[...] += jnp.sum(h * h, axis=0, keepdims=True)

    @pl.when(p == 1)
    def _():
        mu = s1_s[...] / ne
        var = s2_s[...] / ne - mu * mu
        sd = jnp.sqrt(var + EPS)
        ga = ga_ref[...]
        be = be_ref[...]
        msum = jnp.zeros((rb, do), jnp.float32)
        for t in range(3):
            h = h_of(t)
            m = jnp.maximum((h - mu) / sd * ga + be, 0.0)
            msg_ref[pl.Slice(t, rb, 3), :] = m
            msum = msum + m
        xn_ref[...] = msum / 3.0


def _ec_body(x_ref, g_ref, w_ref, b_ref, ga_ref, be_ref, msg_ref, xn_ref,
             e_s, s1_s, s2_s, *, ne, do, de, nb):
    _ec_common(x_ref, g_ref, w_ref, b_ref, ga_ref, be_ref, msg_ref, xn_ref,
               e_s, s1_s, s2_s, ne=ne, do=do, de=de, nb=nb)


def _edge_conv(X, Gn, wT, b_, ga_, be_, de):
    n = X.shape[0]
    do = wT.shape[1]
    rb = RB if n % RB == 0 else n
    nb = n // rb
    body = functools.partial(_ec_body, ne=3 * n, do=do, de=de, nb=nb)
    return pl.pallas_call(
        body,
        grid=(2, nb),
        in_specs=[
            pl.BlockSpec((rb, X.shape[1]), lambda p, b: (b, 0)),
            pl.BlockSpec((3 * rb, Gn.shape[1]), lambda p, b: (b, 0)),
            pl.BlockSpec(wT.shape, lambda p, b: (0, 0)),
            pl.BlockSpec(b_.shape, lambda p, b: (0, 0)),
            pl.BlockSpec(ga_.shape, lambda p, b: (0, 0)),
            pl.BlockSpec(be_.shape, lambda p, b: (0, 0)),
        ],
        out_specs=[
            pl.BlockSpec((3 * rb, do), lambda p, b: (b, 0)),
            pl.BlockSpec((rb, do), lambda p, b: (b, 0)),
        ],
        out_shape=[jax.ShapeDtypeStruct((3 * n, do), jnp.float32),
                   jax.ShapeDtypeStruct((n, do), jnp.float32)],
        scratch_shapes=[pltpu.VMEM((rb, 2 * de), jnp.float32),
                        pltpu.VMEM((1, do), jnp.float32),
                        pltpu.VMEM((1, do), jnp.float32)],
    )(X, Gn, wT, b_, ga_, be_)


def _ec_pool_body(x_ref, g_ref, w_ref, b_ref, ga_ref, be_ref, brow_ref,
                  f1w_ref, f1b_ref, f2w_ref, f2b_ref,
                  msg_ref, xn_ref, out_ref,
                  e_s, s1_s, s2_s, ps_s, cnt_s, *, ne, do, de, nb):
    _ec_common(x_ref, g_ref, w_ref, b_ref, ga_ref, be_ref, msg_ref, xn_ref,
               e_s, s1_s, s2_s, ne=ne, do=do, de=de, nb=nb)
    p = pl.program_id(0)
    b = pl.program_id(1)

    @pl.when(p == 1)
    def _():
        @pl.when(b == 0)
        def _():
            ps_s[...] = jnp.zeros_like(ps_s)
            cnt_s[...] = jnp.zeros_like(cnt_s)
        x2 = xn_ref[...]
        gid = jax.lax.broadcasted_iota(jnp.int32, (G, x2.shape[0]), 0)
        bb_row = brow_ref[...].reshape(1, x2.shape[0])
        oh = jnp.where(gid == bb_row, 1.0, 0.0).astype(jnp.float32)
        ps_s[...] += _mmdot(oh, x2, prec=jax.lax.Precision.HIGHEST)
        cnt_s[...] += jnp.sum(oh, axis=1, keepdims=True)

        @pl.when(b == nb - 1)
        def _():
            pooled = ps_s[...] / jnp.maximum(cnt_s[...], 1.0)
            o1 = _mmdot(pooled, f1w_ref[...]) + f1b_ref[...]
            out_ref[...] = _mmdot(o1, f2w_ref[...]) + f2b_ref[...]


def _edge_conv_pool(X, Gn, wT, b_, ga_, be_, brow_n, f1w, f1b, f2w, f2b, de):
    n = X.shape[0]
    do = wT.shape[1]
    rb = RB if n % RB == 0 else n
    nb = n // rb
    brow_n = brow_n.reshape(nb, 1, rb)
    body = functools.partial(_ec_pool_body, ne=3 * n, do=do, de=de, nb=nb)
    return pl.pallas_call(
        body,
        grid=(2, nb),
        in_specs=[
            pl.BlockSpec((rb, X.shape[1]), lambda p, b: (b, 0)),
            pl.BlockSpec((3 * rb, Gn.shape[1]), lambda p, b: (b, 0)),
            pl.BlockSpec(wT.shape, lambda p, b: (0, 0)),
            pl.BlockSpec(b_.shape, lambda p, b: (0, 0)),
            pl.BlockSpec(ga_.shape, lambda p, b: (0, 0)),
            pl.BlockSpec(be_.shape, lambda p, b: (0, 0)),
            pl.BlockSpec((1, 1, rb), lambda p, b: (b, 0, 0)),
            pl.BlockSpec(f1w.shape, lambda p, b: (0, 0)),
            pl.BlockSpec(f1b.shape, lambda p, b: (0, 0)),
            pl.BlockSpec(f2w.shape, lambda p, b: (0, 0)),
            pl.BlockSpec(f2b.shape, lambda p, b: (0, 0)),
        ],
        out_specs=[
            pl.BlockSpec((3 * rb, do), lambda p, b: (b, 0)),
            pl.BlockSpec((rb, do), lambda p, b: (b, 0)),
            pl.BlockSpec((G, 1), lambda p, b: (0, 0)),
        ],
        out_shape=[jax.ShapeDtypeStruct((3 * n, do), jnp.float32),
                   jax.ShapeDtypeStruct((n, do), jnp.float32),
                   jax.ShapeDtypeStruct((G, 1), jnp.float32)],
        scratch_shapes=[pltpu.VMEM((rb, 2 * de), jnp.float32),
                        pltpu.VMEM((1, do), jnp.float32),
                        pltpu.VMEM((1, do), jnp.float32),
                        pltpu.VMEM((G, do), jnp.float32),
                        pltpu.VMEM((G, 1), jnp.float32)],
    )(X, Gn, wT, b_, ga_, be_, brow_n, f1w, f1b, f2w, f2b)


def kernel(x, batch, w0, b0, g0, be0, w1, b1, g1, be1,
           fc1_w, fc1_b, fc2_w, fc2_b):
    n, dfeat = x.shape
    npad = ((n + T - 1) // T) * T
    nt = npad // T
    e = n * K
    ep = ((e + 255) // 256) * 256

    batch32 = batch.astype(jnp.int32)
    bpad = jnp.concatenate([batch32, jnp.full((npad - n,), G, jnp.int32)])
    brow = bpad.reshape(1, npad)
    bcol = bpad.reshape(npad, 1)

    tile_first = jnp.arange(nt) * T
    tile_last = jnp.minimum(tile_first + T - 1, n - 1)
    g_first = batch32[tile_first][:, None]
    g_last = batch32[tile_last][:, None]
    col_lo = jnp.sum(batch32[None, :] < g_first, axis=1)
    col_hi = jnp.sum(batch32[None, :] <= g_last, axis=1)
    lo_t = (col_lo // T).astype(jnp.int32)
    hi_t = ((col_hi - 1) // T).astype(jnp.int32)

    zpad = jnp.zeros((npad - n, dfeat), jnp.float32)
    xp = jnp.concatenate([x, zpad], axis=0)
    posT = xp[:, :2].T

    tgt = jnp.repeat(jnp.arange(n, dtype=jnp.int32), K)
    idxpad_tail = jnp.zeros((ep - e,), jnp.int32)

    i00, i01, i02 = _knn(xp, posT, brow, bcol, lo_t, hi_t, n, 2)
    src0 = jnp.concatenate([i00, i01, i02], axis=1)[:n].reshape(-1)
    g0rows = _sc_gather(x, jnp.concatenate([src0, idxpad_tail]), 128, 2)
    msg0, x1 = _edge_conv(x, g0rows, w0.T, b0.reshape(1, -1),
                          g0.reshape(1, -1), be0.reshape(1, -1), 128)

    x1p = jnp.concatenate([x1, jnp.zeros((npad - n, 64), jnp.float32)], axis=0)
    x1T = x1p.T
    i10, i11, i12 = _knn(x1p, x1T, brow, bcol, lo_t, hi_t, n, 64)
    src1 = jnp.concatenate([i10, i11, i12], axis=1)[:n].reshape(-1)
    x1w = jnp.concatenate([x1, jnp.zeros((n, 64), jnp.float32)], axis=1)
    g1rows = _sc_gather(x1w, jnp.concatenate([src1, idxpad_tail]), 128, 2)
    msg1, x2, out = _edge_conv_pool(
        x1w, g1rows, w1.T, b1.reshape(1, -1), g1.reshape(1, -1),
        be1.reshape(1, -1), brow[:, :n], fc1_w.T, fc1_b.reshape(1, -1),
        fc2_w.T, fc2_b.reshape(1, -1), 64)

    ea0 = msg0
    ea1 = msg1
    ei0 = jnp.stack([src0, tgt], axis=0).astype(jnp.int64)
    ei1 = jnp.stack([src1, tgt], axis=0).astype(jnp.int64)
    return (out, (ea0, ea1), (x1, x2), (ei0, ei1))


# speedup vs baseline: 1.1618x; 1.1618x over previous
import functools

import jax
import jax.numpy as jnp
from jax.experimental import pallas as pl
from jax.experimental.pallas import tpu as pltpu
from jax.experimental.pallas import tpu_sc as plsc

K = 3
G = 64
T = 512
RB = 2000
EPS = 1e-5
BIG = 1e20
JINF = float("inf")
IBIG = 1 << 30

_PREC = jax.lax.Precision.DEFAULT


def _mmdot(a, b, prec=None):
    return jax.lax.dot_general(
        a, b, (((a.ndim - 1,), (0,)), ((), ())),
        precision=_PREC if prec is None else prec,
        preferred_element_type=jnp.float32)


def _less(a, b):
    return (a[0] < b[0]) | ((a[0] == b[0]) & (a[1] < b[1]))


def _sel(t, a, b):
    return jnp.where(t, a[0], b[0]), jnp.where(t, a[1], b[1])


def _lexmin(a, b):
    return _sel(_less(a, b), a, b)


def _lexmax(a, b):
    return _sel(_less(b, a), a, b)


def _merge3(A, B):
    a0, a1, a2 = A
    b0, b1, b2 = B
    t0 = _less(b0, a0)
    m0 = _sel(t0, b0, a0)
    l0 = _sel(t0, a0, b0)
    opp = _sel(t0, b1, a1)
    m1 = _lexmin(l0, opp)
    c1 = _lexmax(a0, b1)
    c2 = _lexmax(a1, b0)
    m2 = _lexmin(_lexmin(a2, b2), _lexmin(c1, c2))
    return m0, m1, m2


def _knn_body(lo_ref, hi_ref, x_ref, xT_ref, brow_ref, bcol_ref,
              i0_ref, i1_ref, i2_ref, *, n, feat):
    i = pl.program_id(0)
    r0 = i * T
    xr = x_ref[pl.ds(r0, T), 0:feat]
    sqr = jnp.sum(xr * xr, axis=1, keepdims=True)
    bcr = bcol_ref[pl.ds(r0, T), :]
    rowid = r0 + jax.lax.broadcasted_iota(jnp.int32, (T, 1), 0)
    lo = lo_ref[i]
    hi = hi_ref[i]

    def col_tile(j, carry):
        v0, i0, v1, i1, v2, i2 = carry
        c0 = j * T
        xc = xT_ref[:, pl.ds(c0, T)]
        sqc = jnp.sum(xc * xc, axis=0, keepdims=True)
        dot = _mmdot(xr, xc)
        d2 = sqr + sqc - 2.0 * dot
        bcc = brow_ref[:, pl.ds(c0, T)]
        colid = c0 + jax.lax.broadcasted_iota(jnp.int32, (T, T), 1)
        d2 = jnp.where((bcr != bcc) | (rowid == colid), BIG, d2)
        d2 = jnp.where(colid >= n, JINF, d2)
        tile = []
        d = d2
        for s in range(3):
            mv = jnp.min(d, axis=1, keepdims=True)
            mi = jnp.min(jnp.where(d == mv, colid, IBIG), axis=1, keepdims=True)
            tile.append((mv, mi))
            if s < 2:
                d = jnp.where(colid == mi, JINF, d)
        (v0, i0), (v1, i1), (v2, i2) = _merge3(
            ((v0, i0), (v1, i1), (v2, i2)), tuple(tile))
        return v0, i0, v1, i1, v2, i2

    finf = jnp.full((T, 1), JINF, jnp.float32)
    init = (finf, jnp.full((T, 1), IBIG, jnp.int32),
            finf, jnp.full((T, 1), IBIG + 1, jnp.int32),
            finf, jnp.full((T, 1), IBIG + 2, jnp.int32))
    v0, i0, v1, i1, v2, i2 = jax.lax.fori_loop(lo, hi + 1, col_tile, init)

    hc = (hi + 1) * T
    base = jnp.where(lo > 0, 0, hc)
    ones_f = jnp.ones((T, 1), jnp.float32)
    ones_i = jnp.ones((T, 1), jnp.int32)
    virt = []
    for s in range(3):
        vidx = base + s
        vval = jnp.where(vidx < n, BIG, JINF)
        virt.append((vval * ones_f, vidx * ones_i))
    (v0, i0), (v1, i1), (v2, i2) = _merge3(
        ((v0, i0), (v1, i1), (v2, i2)), tuple(virt))

    i0_ref[...] = i0
    i1_ref[...] = i1
    i2_ref[...] = i2


def _knn(xp, xT, brow, bcol, lo_t, hi_t, n, feat):
    npad = xp.shape[0]
    nt = npad // T
    body = functools.partial(_knn_body, n=n, feat=feat)
    grid_spec = pltpu.PrefetchScalarGridSpec(
        num_scalar_prefetch=2,
        grid=(nt,),
        in_specs=[
            pl.BlockSpec(xp.shape, lambda i, *_: (0, 0)),
            pl.BlockSpec(xT.shape, lambda i, *_: (0, 0)),
            pl.BlockSpec(brow.shape, lambda i, *_: (0, 0)),
            pl.BlockSpec(bcol.shape, lambda i, *_: (0, 0)),
        ],
        out_specs=[
            pl.BlockSpec((T, 1), lambda i, *_: (i, 0)),
            pl.BlockSpec((T, 1), lambda i, *_: (i, 0)),
            pl.BlockSpec((T, 1), lambda i, *_: (i, 0)),
        ],
    )
    out_shape = [jax.ShapeDtypeStruct((npad, 1), jnp.int32)] * 3
    return pl.pallas_call(
        body, grid_spec=grid_spec, out_shape=out_shape,
        compiler_params=pltpu.CompilerParams(
            dimension_semantics=("parallel",)),
    )(lo_t, hi_t, xp, xT, brow, bcol)


def _sc_gather(table, idx, width, chunks):
    nc, ns = 2, 16
    nw = nc * ns
    ep = idx.shape[0]
    bpw = ep // nw
    cs = bpw // chunks
    mesh = plsc.VectorSubcoreMesh(core_axis_name="c", subcore_axis_name="s")

    @functools.partial(
        pl.kernel, mesh=mesh,
        out_type=jax.ShapeDtypeStruct((ep, width), jnp.float32),
        scratch_types=[
            pltpu.VMEM((bpw,), jnp.int32),
            pltpu.VMEM((cs, width), jnp.float32),
            pltpu.VMEM((cs, width), jnp.float32),
            pltpu.SemaphoreType.DMA,
            pltpu.SemaphoreType.DMA,
        ])
    def k(table_hbm, idx_hbm, out_hbm, idx_v, rows_a, rows_b, sem_a, sem_b):
        wid = jax.lax.axis_index("s") * nc + jax.lax.axis_index("c")
        base = wid * bpw
        pltpu.sync_copy(idx_hbm.at[pl.ds(base, bpw)], idx_v)
        bufs = [(rows_a, sem_a), (rows_b, sem_b)]
        copies = []
        for c in range(chunks):
            src = idx_v if chunks == 1 else idx_v.at[pl.ds(c * cs, cs)]
            rows_v, sem = bufs[c % 2]
            copies.append(pltpu.async_copy(table_hbm.at[src], rows_v, sem))
        for c in range(chunks):
            rows_v, _ = bufs[c % 2]
            copies[c].wait()
            pltpu.sync_copy(rows_v, out_hbm.at[pl.ds(base + c * cs, cs)])

    return k(table, idx)


def _ec_common(x_ref, g_ref, w_ref, b_ref, ga_ref, be_ref, msg_ref, xn_ref,
               e_s, s1_s, s2_s, *, ne, do, de, nb):
    p = pl.program_id(0)
    b = pl.program_id(1)
    rb = x_ref.shape[0]
    e_s[:, 0:de] = x_ref[:, 0:de]

    def h_of(t):
        e_s[:, de:2 * de] = g_ref[pl.Slice(t, rb, 3), 0:de]
        return _mmdot(e_s[...], w_ref[...]) + b_ref[...]

    @pl.when(p == 0)
    def _():
        @pl.when(b == 0)
        def _():
            s1_s[...] = jnp.zeros_like(s1_s)
            s2_s[...] = jnp.zeros_like(s2_s)
        for t in range(3):
            h = h_of(t)
            s1_s[...] += jnp.sum(h, axis=0, keepdims=True)
            s2_s[...] += jnp.sum(h * h, axis=0, keepdims=True)

    @pl.when(p == 1)
    def _():
        mu = s1_s[...] / ne
        var = s2_s[...] / ne - mu * mu
        sd = jnp.sqrt(var + EPS)
        ga = ga_ref[...]
        be = be_ref[...]
        msum = jnp.zeros((rb, do), jnp.float32)
        for t in range(3):
            h = h_of(t)
            m = jnp.maximum((h - mu) / sd * ga + be, 0.0)
            msg_ref[pl.Slice(t, rb, 3), :] = m
            msum = msum + m
        xn_ref[...] = msum / 3.0


def _ec_body(x_ref, g_ref, w_ref, b_ref, ga_ref, be_ref, msg_ref, xn_ref,
             e_s, s1_s, s2_s, *, ne, do, de, nb):
    _ec_common(x_ref, g_ref, w_ref, b_ref, ga_ref, be_ref, msg_ref, xn_ref,
               e_s, s1_s, s2_s, ne=ne, do=do, de=de, nb=nb)


def _edge_conv(X, Gn, wT, b_, ga_, be_, de):
    n = X.shape[0]
    do = wT.shape[1]
    rb = RB if n % RB == 0 else n
    nb = n // rb
    body = functools.partial(_ec_body, ne=3 * n, do=do, de=de, nb=nb)
    return pl.pallas_call(
        body,
        grid=(2, nb),
        in_specs=[
            pl.BlockSpec((rb, X.shape[1]), lambda p, b: (b, 0)),
            pl.BlockSpec((3 * rb, Gn.shape[1]), lambda p, b: (b, 0)),
            pl.BlockSpec(wT.shape, lambda p, b: (0, 0)),
            pl.BlockSpec(b_.shape, lambda p, b: (0, 0)),
            pl.BlockSpec(ga_.shape, lambda p, b: (0, 0)),
            pl.BlockSpec(be_.shape, lambda p, b: (0, 0)),
        ],
        out_specs=[
            pl.BlockSpec((3 * rb, do), lambda p, b: (b, 0)),
            pl.BlockSpec((rb, do), lambda p, b: (b, 0)),
        ],
        out_shape=[jax.ShapeDtypeStruct((3 * n, do), jnp.float32),
                   jax.ShapeDtypeStruct((n, do), jnp.float32)],
        scratch_shapes=[pltpu.VMEM((rb, 2 * de), jnp.float32),
                        pltpu.VMEM((1, do), jnp.float32),
                        pltpu.VMEM((1, do), jnp.float32)],
    )(X, Gn, wT, b_, ga_, be_)


def _ec_pool_body(x_ref, g_ref, w_ref, b_ref, ga_ref, be_ref, brow_ref,
                  f1w_ref, f1b_ref, f2w_ref, f2b_ref,
                  msg_ref, xn_ref, out_ref,
                  e_s, s1_s, s2_s, ps_s, cnt_s, *, ne, do, de, nb):
    _ec_common(x_ref, g_ref, w_ref, b_ref, ga_ref, be_ref, msg_ref, xn_ref,
               e_s, s1_s, s2_s, ne=ne, do=do, de=de, nb=nb)
    p = pl.program_id(0)
    b = pl.program_id(1)

    @pl.when(p == 1)
    def _():
        @pl.when(b == 0)
        def _():
            ps_s[...] = jnp.zeros_like(ps_s)
            cnt_s[...] = jnp.zeros_like(cnt_s)
        x2 = xn_ref[...]
        gid = jax.lax.broadcasted_iota(jnp.int32, (G, x2.shape[0]), 0)
        bb_row = brow_ref[...].reshape(1, x2.shape[0])
        oh = jnp.where(gid == bb_row, 1.0, 0.0).astype(jnp.float32)
        ps_s[...] += _mmdot(oh, x2, prec=jax.lax.Precision.HIGHEST)
        cnt_s[...] += jnp.sum(oh, axis=1, keepdims=True)

        @pl.when(b == nb - 1)
        def _():
            pooled = ps_s[...] / jnp.maximum(cnt_s[...], 1.0)
            o1 = _mmdot(pooled, f1w_ref[...]) + f1b_ref[...]
            out_ref[...] = _mmdot(o1, f2w_ref[...]) + f2b_ref[...]


def _edge_conv_pool(X, Gn, wT, b_, ga_, be_, brow_n, f1w, f1b, f2w, f2b, de):
    n = X.shape[0]
    do = wT.shape[1]
    rb = RB if n % RB == 0 else n
    nb = n // rb
    brow_n = brow_n.reshape(nb, 1, rb)
    body = functools.partial(_ec_pool_body, ne=3 * n, do=do, de=de, nb=nb)
    return pl.pallas_call(
        body,
        grid=(2, nb),
        in_specs=[
            pl.BlockSpec((rb, X.shape[1]), lambda p, b: (b, 0)),
            pl.BlockSpec((3 * rb, Gn.shape[1]), lambda p, b: (b, 0)),
            pl.BlockSpec(wT.shape, lambda p, b: (0, 0)),
            pl.BlockSpec(b_.shape, lambda p, b: (0, 0)),
            pl.BlockSpec(ga_.shape, lambda p, b: (0, 0)),
            pl.BlockSpec(be_.shape, lambda p, b: (0, 0)),
            pl.BlockSpec((1, 1, rb), lambda p, b: (b, 0, 0)),
            pl.BlockSpec(f1w.shape, lambda p, b: (0, 0)),
            pl.BlockSpec(f1b.shape, lambda p, b: (0, 0)),
            pl.BlockSpec(f2w.shape, lambda p, b: (0, 0)),
            pl.BlockSpec(f2b.shape, lambda p, b: (0, 0)),
        ],
        out_specs=[
            pl.BlockSpec((3 * rb, do), lambda p, b: (b, 0)),
            pl.BlockSpec((rb, do), lambda p, b: (b, 0)),
            pl.BlockSpec((G, 1), lambda p, b: (0, 0)),
        ],
        out_shape=[jax.ShapeDtypeStruct((3 * n, do), jnp.float32),
                   jax.ShapeDtypeStruct((n, do), jnp.float32),
                   jax.ShapeDtypeStruct((G, 1), jnp.float32)],
        scratch_shapes=[pltpu.VMEM((rb, 2 * de), jnp.float32),
                        pltpu.VMEM((1, do), jnp.float32),
                        pltpu.VMEM((1, do), jnp.float32),
                        pltpu.VMEM((G, do), jnp.float32),
                        pltpu.VMEM((G, 1), jnp.float32)],
    )(X, Gn, wT, b_, ga_, be_, brow_n, f1w, f1b, f2w, f2b)


def kernel(x, batch, w0, b0, g0, be0, w1, b1, g1, be1,
           fc1_w, fc1_b, fc2_w, fc2_b):
    n, dfeat = x.shape
    npad = ((n + T - 1) // T) * T
    nt = npad // T
    e = n * K
    ep = ((e + 255) // 256) * 256

    batch32 = batch.astype(jnp.int32)
    bpad = jnp.concatenate([batch32, jnp.full((npad - n,), G, jnp.int32)])
    brow = bpad.reshape(1, npad)
    bcol = bpad.reshape(npad, 1)

    tile_first = jnp.arange(nt) * T
    tile_last = jnp.minimum(tile_first + T - 1, n - 1)
    g_first = batch32[tile_first][:, None]
    g_last = batch32[tile_last][:, None]
    col_lo = jnp.sum(batch32[None, :] < g_first, axis=1)
    col_hi = jnp.sum(batch32[None, :] <= g_last, axis=1)
    lo_t = (col_lo // T).astype(jnp.int32)
    hi_t = ((col_hi - 1) // T).astype(jnp.int32)

    zpad = jnp.zeros((npad - n, dfeat), jnp.float32)
    xp = jnp.concatenate([x, zpad], axis=0)
    posT = xp[:, :2].T

    tgt = jnp.repeat(jnp.arange(n, dtype=jnp.int32), K)
    idxpad_tail = jnp.zeros((ep - e,), jnp.int32)

    i00, i01, i02 = _knn(xp, posT, brow, bcol, lo_t, hi_t, n, 2)
    src0 = jnp.concatenate([i00, i01, i02], axis=1)[:n].reshape(-1)
    g0rows = _sc_gather(x, jnp.concatenate([src0, idxpad_tail]), 128, 2)
    msg0, x1 = _edge_conv(x, g0rows, w0.T, b0.reshape(1, -1),
                          g0.reshape(1, -1), be0.reshape(1, -1), 128)

    x1p = jnp.concatenate([x1, jnp.zeros((npad - n, 64), jnp.float32)], axis=0)
    x1T = x1p.T
    i10, i11, i12 = _knn(x1p, x1T, brow, bcol, lo_t, hi_t, n, 64)
    src1 = jnp.concatenate([i10, i11, i12], axis=1)[:n].reshape(-1)
    x1w = jnp.concatenate([x1, jnp.zeros((n, 64), jnp.float32)], axis=1)
    g1rows = _sc_gather(x1w, jnp.concatenate([src1, idxpad_tail]), 128, 2)
    msg1, x2, out = _edge_conv_pool(
        x1w, g1rows, w1.T, b1.reshape(1, -1), g1.reshape(1, -1),
        be1.reshape(1, -1), brow[:, :n], fc1_w.T, fc1_b.reshape(1, -1),
        fc2_w.T, fc2_b.reshape(1, -1), 64)

    ea0 = msg0
    ea1 = msg1
    ei0 = jnp.stack([src0, tgt], axis=0).astype(jnp.int64)
    ei1 = jnp.stack([src1, tgt], axis=0).astype(jnp.int64)
    return (out, (ea0, ea1), (x1, x2), (ei0, ei1))


# speedup vs baseline: 1.1699x; 1.0070x over previous
import functools

import jax
import jax.numpy as jnp
from jax.experimental import pallas as pl
from jax.experimental.pallas import tpu as pltpu
from jax.experimental.pallas import tpu_sc as plsc

K = 3
G = 64
T = 512
RB = 2000
EPS = 1e-5
BIG = 1e20
JINF = float("inf")
IBIG = 1 << 30

_PREC = jax.lax.Precision.DEFAULT


def _mmdot(a, b, prec=None):
    return jax.lax.dot_general(
        a, b, (((a.ndim - 1,), (0,)), ((), ())),
        precision=_PREC if prec is None else prec,
        preferred_element_type=jnp.float32)


def _less(a, b):
    return (a[0] < b[0]) | ((a[0] == b[0]) & (a[1] < b[1]))


def _sel(t, a, b):
    return jnp.where(t, a[0], b[0]), jnp.where(t, a[1], b[1])


def _lexmin(a, b):
    return _sel(_less(a, b), a, b)


def _lexmax(a, b):
    return _sel(_less(b, a), a, b)


def _merge3(A, B):
    a0, a1, a2 = A
    b0, b1, b2 = B
    t0 = _less(b0, a0)
    m0 = _sel(t0, b0, a0)
    l0 = _sel(t0, a0, b0)
    opp = _sel(t0, b1, a1)
    m1 = _lexmin(l0, opp)
    c1 = _lexmax(a0, b1)
    c2 = _lexmax(a1, b0)
    m2 = _lexmin(_lexmin(a2, b2), _lexmin(c1, c2))
    return m0, m1, m2


def _knn_body(lo_ref, hi_ref, x_ref, xT2_ref, brow_ref, bcol_ref,
              i0_ref, i1_ref, i2_ref, *, n, feat):
    i = pl.program_id(0)
    r0 = i * T
    xr = x_ref[pl.ds(r0, T), 0:feat]
    sqr = jnp.sum(xr * xr, axis=1, keepdims=True)
    bcr = bcol_ref[pl.ds(r0, T), :]
    rowid = r0 + jax.lax.broadcasted_iota(jnp.int32, (T, 1), 0)
    lo = lo_ref[i]
    hi = hi_ref[i]

    def col_tile(j, carry):
        v0, i0, v1, i1, v2, i2 = carry
        c0 = j * T
        xc2 = xT2_ref[:, pl.ds(c0, T)]
        sqc = jnp.sum(xc2 * xc2, axis=0, keepdims=True) * 0.25
        dot2 = _mmdot(xr, xc2)
        d2 = (sqr + sqc) - dot2
        bcc = brow_ref[:, pl.ds(c0, T)]
        colid = c0 + jax.lax.broadcasted_iota(jnp.int32, (T, T), 1)
        d2 = jnp.where((bcr != bcc) | (rowid == colid), BIG, d2)
        d2 = jnp.where(colid >= n, JINF, d2)
        tile = []
        d = d2
        for s in range(3):
            mv = jnp.min(d, axis=1, keepdims=True)
            mi = jnp.min(jnp.where(d == mv, colid, IBIG), axis=1, keepdims=True)
            tile.append((mv, mi))
            if s < 2:
                d = jnp.where(colid == mi, JINF, d)
        (v0, i0), (v1, i1), (v2, i2) = _merge3(
            ((v0, i0), (v1, i1), (v2, i2)), tuple(tile))
        return v0, i0, v1, i1, v2, i2

    finf = jnp.full((T, 1), JINF, jnp.float32)
    init = (finf, jnp.full((T, 1), IBIG, jnp.int32),
            finf, jnp.full((T, 1), IBIG + 1, jnp.int32),
            finf, jnp.full((T, 1), IBIG + 2, jnp.int32))
    v0, i0, v1, i1, v2, i2 = jax.lax.fori_loop(lo, hi + 1, col_tile, init)

    hc = (hi + 1) * T
    base = jnp.where(lo > 0, 0, hc)
    ones_f = jnp.ones((T, 1), jnp.float32)
    ones_i = jnp.ones((T, 1), jnp.int32)
    virt = []
    for s in range(3):
        vidx = base + s
        vval = jnp.where(vidx < n, BIG, JINF)
        virt.append((vval * ones_f, vidx * ones_i))
    (v0, i0), (v1, i1), (v2, i2) = _merge3(
        ((v0, i0), (v1, i1), (v2, i2)), tuple(virt))

    i0_ref[...] = i0
    i1_ref[...] = i1
    i2_ref[...] = i2


def _knn(xp, xT2, brow, bcol, lo_t, hi_t, n, feat):
    npad = xp.shape[0]
    nt = npad // T
    body = functools.partial(_knn_body, n=n, feat=feat)
    grid_spec = pltpu.PrefetchScalarGridSpec(
        num_scalar_prefetch=2,
        grid=(nt,),
        in_specs=[
            pl.BlockSpec(xp.shape, lambda i, *_: (0, 0)),
            pl.BlockSpec(xT2.shape, lambda i, *_: (0, 0)),
            pl.BlockSpec(brow.shape, lambda i, *_: (0, 0)),
            pl.BlockSpec(bcol.shape, lambda i, *_: (0, 0)),
        ],
        out_specs=[
            pl.BlockSpec((T, 1), lambda i, *_: (i, 0)),
            pl.BlockSpec((T, 1), lambda i, *_: (i, 0)),
            pl.BlockSpec((T, 1), lambda i, *_: (i, 0)),
        ],
    )
    out_shape = [jax.ShapeDtypeStruct((npad, 1), jnp.int32)] * 3
    return pl.pallas_call(
        body, grid_spec=grid_spec, out_shape=out_shape,
        compiler_params=pltpu.CompilerParams(
            dimension_semantics=("parallel",)),
    )(lo_t, hi_t, xp, xT2, brow, bcol)


def _sc_gather(table, idx, width, chunks):
    nc, ns = 2, 16
    nw = nc * ns
    ep = idx.shape[0]
    bpw = ep // nw
    cs = bpw // chunks
    mesh = plsc.VectorSubcoreMesh(core_axis_name="c", subcore_axis_name="s")

    @functools.partial(
        pl.kernel, mesh=mesh,
        out_type=jax.ShapeDtypeStruct((ep, width), jnp.float32),
        scratch_types=[
            pltpu.VMEM((bpw,), jnp.int32),
            pltpu.VMEM((cs, width), jnp.float32),
            pltpu.VMEM((cs, width), jnp.float32),
            pltpu.SemaphoreType.DMA,
            pltpu.SemaphoreType.DMA,
        ])
    def k(table_hbm, idx_hbm, out_hbm, idx_v, rows_a, rows_b, sem_a, sem_b):
        wid = jax.lax.axis_index("s") * nc + jax.lax.axis_index("c")
        base = wid * bpw
        pltpu.sync_copy(idx_hbm.at[pl.ds(base, bpw)], idx_v)
        bufs = [(rows_a, sem_a), (rows_b, sem_b)]
        copies = []
        for c in range(chunks):
            src = idx_v if chunks == 1 else idx_v.at[pl.ds(c * cs, cs)]
            rows_v, sem = bufs[c % 2]
            copies.append(pltpu.async_copy(table_hbm.at[src], rows_v, sem))
        for c in range(chunks):
            rows_v, _ = bufs[c % 2]
            copies[c].wait()
            pltpu.sync_copy(rows_v, out_hbm.at[pl.ds(base + c * cs, cs)])

    return k(table, idx)


def _ec_common(x_ref, g_ref, w_ref, b_ref, ga_ref, be_ref, msg_ref, xn_ref,
               e_s, s1_s, s2_s, *, ne, do, de, nb):
    p = pl.program_id(0)
    b = pl.program_id(1)
    rb = x_ref.shape[0]
    e_s[:, 0:de] = x_ref[:, 0:de]

    def h_of(t):
        e_s[:, de:2 * de] = g_ref[pl.Slice(t, rb, 3), 0:de]
        return _mmdot(e_s[...], w_ref[...]) + b_ref[...]

    @pl.when(p == 0)
    def _():
        @pl.when(b == 0)
        def _():
            s1_s[...] = jnp.zeros_like(s1_s)
            s2_s[...] = jnp.zeros_like(s2_s)
        for t in range(3):
            h = h_of(t)
            s1_s[...] += jnp.sum(h, axis=0, keepdims=True)
            s2_s[...] += jnp.sum(h * h, axis=0, keepdims=True)

    @pl.when(p == 1)
    def _():
        mu = s1_s[...] / ne
        var = s2_s[...] / ne - mu * mu
        sd = jnp.sqrt(var + EPS)
        ga = ga_ref[...]
        be = be_ref[...]
        msum = jnp.zeros((rb, do), jnp.float32)
        for t in range(3):
            h = h_of(t)
            m = jnp.maximum((h - mu) / sd * ga + be, 0.0)
            msg_ref[pl.Slice(t, rb, 3), :] = m
            msum = msum + m
        xn_ref[...] = msum / 3.0


def _ec_body(x_ref, g_ref, w_ref, b_ref, ga_ref, be_ref, msg_ref, xn_ref,
             e_s, s1_s, s2_s, *, ne, do, de, nb):
    _ec_common(x_ref, g_ref, w_ref, b_ref, ga_ref, be_ref, msg_ref, xn_ref,
               e_s, s1_s, s2_s, ne=ne, do=do, de=de, nb=nb)


def _edge_conv(X, Gn, wT, b_, ga_, be_, de):
    n = X.shape[0]
    do = wT.shape[1]
    rb = RB if n % RB == 0 else n
    nb = n // rb
    body = functools.partial(_ec_body, ne=3 * n, do=do, de=de, nb=nb)
    return pl.pallas_call(
        body,
        grid=(2, nb),
        in_specs=[
            pl.BlockSpec((rb, X.shape[1]), lambda p, b: (b, 0)),
            pl.BlockSpec((3 * rb, Gn.shape[1]), lambda p, b: (b, 0)),
            pl.BlockSpec(wT.shape, lambda p, b: (0, 0)),
            pl.BlockSpec(b_.shape, lambda p, b: (0, 0)),
            pl.BlockSpec(ga_.shape, lambda p, b: (0, 0)),
            pl.BlockSpec(be_.shape, lambda p, b: (0, 0)),
        ],
        out_specs=[
            pl.BlockSpec((3 * rb, do), lambda p, b: (b, 0)),
            pl.BlockSpec((rb, do), lambda p, b: (b, 0)),
        ],
        out_shape=[jax.ShapeDtypeStruct((3 * n, do), jnp.float32),
                   jax.ShapeDtypeStruct((n, do), jnp.float32)],
        scratch_shapes=[pltpu.VMEM((rb, 2 * de), jnp.float32),
                        pltpu.VMEM((1, do), jnp.float32),
                        pltpu.VMEM((1, do), jnp.float32)],
    )(X, Gn, wT, b_, ga_, be_)


def _ec_pool_body(x_ref, g_ref, w_ref, b_ref, ga_ref, be_ref, brow_ref,
                  f1w_ref, f1b_ref, f2w_ref, f2b_ref,
                  msg_ref, xn_ref, out_ref,
                  e_s, s1_s, s2_s, ps_s, cnt_s, *, ne, do, de, nb):
    _ec_common(x_ref, g_ref, w_ref, b_ref, ga_ref, be_ref, msg_ref, xn_ref,
               e_s, s1_s, s2_s, ne=ne, do=do, de=de, nb=nb)
    p = pl.program_id(0)
    b = pl.program_id(1)

    @pl.when(p == 1)
    def _():
        @pl.when(b == 0)
        def _():
            ps_s[...] = jnp.zeros_like(ps_s)
            cnt_s[...] = jnp.zeros_like(cnt_s)
        x2 = xn_ref[...]
        gid = jax.lax.broadcasted_iota(jnp.int32, (G, x2.shape[0]), 0)
        bb_row = brow_ref[...].reshape(1, x2.shape[0])
        oh = jnp.where(gid == bb_row, 1.0, 0.0).astype(jnp.float32)
        ps_s[...] += _mmdot(oh, x2, prec=jax.lax.Precision.HIGHEST)
        cnt_s[...] += jnp.sum(oh, axis=1, keepdims=True)

        @pl.when(b == nb - 1)
        def _():
            pooled = ps_s[...] / jnp.maximum(cnt_s[...], 1.0)
            o1 = _mmdot(pooled, f1w_ref[...]) + f1b_ref[...]
            out_ref[...] = _mmdot(o1, f2w_ref[...]) + f2b_ref[...]


def _edge_conv_pool(X, Gn, wT, b_, ga_, be_, brow_n, f1w, f1b, f2w, f2b, de):
    n = X.shape[0]
    do = wT.shape[1]
    rb = RB if n % RB == 0 else n
    nb = n // rb
    brow_n = brow_n.reshape(nb, 1, rb)
    body = functools.partial(_ec_pool_body, ne=3 * n, do=do, de=de, nb=nb)
    return pl.pallas_call(
        body,
        grid=(2, nb),
        in_specs=[
            pl.BlockSpec((rb, X.shape[1]), lambda p, b: (b, 0)),
            pl.BlockSpec((3 * rb, Gn.shape[1]), lambda p, b: (b, 0)),
            pl.BlockSpec(wT.shape, lambda p, b: (0, 0)),
            pl.BlockSpec(b_.shape, lambda p, b: (0, 0)),
            pl.BlockSpec(ga_.shape, lambda p, b: (0, 0)),
            pl.BlockSpec(be_.shape, lambda p, b: (0, 0)),
            pl.BlockSpec((1, 1, rb), lambda p, b: (b, 0, 0)),
            pl.BlockSpec(f1w.shape, lambda p, b: (0, 0)),
            pl.BlockSpec(f1b.shape, lambda p, b: (0, 0)),
            pl.BlockSpec(f2w.shape, lambda p, b: (0, 0)),
            pl.BlockSpec(f2b.shape, lambda p, b: (0, 0)),
        ],
        out_specs=[
            pl.BlockSpec((3 * rb, do), lambda p, b: (b, 0)),
            pl.BlockSpec((rb, do), lambda p, b: (b, 0)),
            pl.BlockSpec((G, 1), lambda p, b: (0, 0)),
        ],
        out_shape=[jax.ShapeDtypeStruct((3 * n, do), jnp.float32),
                   jax.ShapeDtypeStruct((n, do), jnp.float32),
                   jax.ShapeDtypeStruct((G, 1), jnp.float32)],
        scratch_shapes=[pltpu.VMEM((rb, 2 * de), jnp.float32),
                        pltpu.VMEM((1, do), jnp.float32),
                        pltpu.VMEM((1, do), jnp.float32),
                        pltpu.VMEM((G, do), jnp.float32),
                        pltpu.VMEM((G, 1), jnp.float32)],
    )(X, Gn, wT, b_, ga_, be_, brow_n, f1w, f1b, f2w, f2b)


def kernel(x, batch, w0, b0, g0, be0, w1, b1, g1, be1,
           fc1_w, fc1_b, fc2_w, fc2_b):
    n, dfeat = x.shape
    npad = ((n + T - 1) // T) * T
    nt = npad // T
    e = n * K
    ep = ((e + 255) // 256) * 256

    batch32 = batch.astype(jnp.int32)
    bpad = jnp.concatenate([batch32, jnp.full((npad - n,), G, jnp.int32)])
    brow = bpad.reshape(1, npad)
    bcol = bpad.reshape(npad, 1)

    tile_first = jnp.arange(nt) * T
    tile_last = jnp.minimum(tile_first + T - 1, n - 1)
    g_first = batch32[tile_first][:, None]
    g_last = batch32[tile_last][:, None]
    col_lo = jnp.sum(batch32[None, :] < g_first, axis=1)
    col_hi = jnp.sum(batch32[None, :] <= g_last, axis=1)
    lo_t = (col_lo // T).astype(jnp.int32)
    hi_t = ((col_hi - 1) // T).astype(jnp.int32)

    zpad = jnp.zeros((npad - n, dfeat), jnp.float32)
    xp = jnp.concatenate([x, zpad], axis=0)
    posT = xp[:, :2].T

    tgt = jnp.repeat(jnp.arange(n, dtype=jnp.int32), K)
    idxpad_tail = jnp.zeros((ep - e,), jnp.int32)

    i00, i01, i02 = _knn(xp, 2.0 * posT, brow, bcol, lo_t, hi_t, n, 2)
    src0 = jnp.concatenate([i00, i01, i02], axis=1)[:n].reshape(-1)
    g0rows = _sc_gather(x, jnp.concatenate([src0, idxpad_tail]), 128, 2)
    msg0, x1 = _edge_conv(x, g0rows, w0.T, b0.reshape(1, -1),
                          g0.reshape(1, -1), be0.reshape(1, -1), 128)

    x1p = jnp.concatenate([x1, jnp.zeros((npad - n, 64), jnp.float32)], axis=0)
    x1T2 = 2.0 * x1p.T
    i10, i11, i12 = _knn(x1p, x1T2, brow, bcol, lo_t, hi_t, n, 64)
    src1 = jnp.concatenate([i10, i11, i12], axis=1)[:n].reshape(-1)
    x1w = jnp.concatenate([x1, jnp.zeros((n, 64), jnp.float32)], axis=1)
    g1rows = _sc_gather(x1w, jnp.concatenate([src1, idxpad_tail]), 128, 2)
    msg1, x2, out = _edge_conv_pool(
        x1w, g1rows, w1.T, b1.reshape(1, -1), g1.reshape(1, -1),
        be1.reshape(1, -1), brow[:, :n], fc1_w.T, fc1_b.reshape(1, -1),
        fc2_w.T, fc2_b.reshape(1, -1), 64)

    ea0 = msg0
    ea1 = msg1
    ei0 = jnp.stack([src0, tgt], axis=0).astype(jnp.int64)
    ei1 = jnp.stack([src1, tgt], axis=0).astype(jnp.int64)
    return (out, (ea0, ea1), (x1, x2), (ei0, ei1))


# speedup vs baseline: 1.1980x; 1.0240x over previous
import functools

import numpy as np

import jax
import jax.numpy as jnp
from jax.experimental import pallas as pl
from jax.experimental.pallas import tpu as pltpu
from jax.experimental.pallas import tpu_sc as plsc

K = 3
G = 64
T = 512
RB = 2000
EPS = 1e-5
BIG = 1e20
JINF = float("inf")
IBIG = 1 << 30

_PREC = jax.lax.Precision.DEFAULT


def _mmdot(a, b, prec=None):
    return jax.lax.dot_general(
        a, b, (((a.ndim - 1,), (0,)), ((), ())),
        precision=_PREC if prec is None else prec,
        preferred_element_type=jnp.float32)


def _less(a, b):
    return (a[0] < b[0]) | ((a[0] == b[0]) & (a[1] < b[1]))


def _sel(t, a, b):
    return jnp.where(t, a[0], b[0]), jnp.where(t, a[1], b[1])


def _lexmin(a, b):
    return _sel(_less(a, b), a, b)


def _lexmax(a, b):
    return _sel(_less(b, a), a, b)


def _merge3(A, B):
    a0, a1, a2 = A
    b0, b1, b2 = B
    t0 = _less(b0, a0)
    m0 = _sel(t0, b0, a0)
    l0 = _sel(t0, a0, b0)
    opp = _sel(t0, b1, a1)
    m1 = _lexmin(l0, opp)
    c1 = _lexmax(a0, b1)
    c2 = _lexmax(a1, b0)
    m2 = _lexmin(_lexmin(a2, b2), _lexmin(c1, c2))
    return m0, m1, m2


def _knn_body(lo_ref, hi_ref, x_ref, xT2_ref, brow_ref, bcol_ref,
              i0_ref, i1_ref, i2_ref, *, n, feat):
    i = pl.program_id(0)
    r0 = i * T
    xr = x_ref[pl.ds(r0, T), 0:feat]
    sqr = jnp.sum(xr * xr, axis=1, keepdims=True)
    bcr = bcol_ref[pl.ds(r0, T), :]
    rowid = r0 + jax.lax.broadcasted_iota(jnp.int32, (T, 1), 0)
    lo = lo_ref[i]
    hi = hi_ref[i]

    def col_tile(j, carry):
        v0, i0, v1, i1, v2, i2 = carry
        c0 = j * T
        xc2 = xT2_ref[:, pl.ds(c0, T)]
        sqc = jnp.sum(xc2 * xc2, axis=0, keepdims=True) * 0.25
        dot2 = _mmdot(xr, xc2)
        d2 = (sqr + sqc) - dot2
        bcc = brow_ref[:, pl.ds(c0, T)]
        colid = c0 + jax.lax.broadcasted_iota(jnp.int32, (T, T), 1)
        d2 = jnp.where((bcr != bcc) | (rowid == colid), BIG, d2)
        d2 = jnp.where(colid >= n, JINF, d2)
        tile = []
        d = d2
        for s in range(3):
            mv = jnp.min(d, axis=1, keepdims=True)
            mi = jnp.min(jnp.where(d == mv, colid, IBIG), axis=1, keepdims=True)
            tile.append((mv, mi))
            if s < 2:
                d = jnp.where(colid == mi, JINF, d)
        (v0, i0), (v1, i1), (v2, i2) = _merge3(
            ((v0, i0), (v1, i1), (v2, i2)), tuple(tile))
        return v0, i0, v1, i1, v2, i2

    finf = jnp.full((T, 1), JINF, jnp.float32)
    init = (finf, jnp.full((T, 1), IBIG, jnp.int32),
            finf, jnp.full((T, 1), IBIG + 1, jnp.int32),
            finf, jnp.full((T, 1), IBIG + 2, jnp.int32))
    v0, i0, v1, i1, v2, i2 = jax.lax.fori_loop(lo, hi + 1, col_tile, init)

    hc = (hi + 1) * T
    base = jnp.where(lo > 0, 0, hc)
    ones_f = jnp.ones((T, 1), jnp.float32)
    ones_i = jnp.ones((T, 1), jnp.int32)
    virt = []
    for s in range(3):
        vidx = base + s
        vval = jnp.where(vidx < n, BIG, JINF)
        virt.append((vval * ones_f, vidx * ones_i))
    (v0, i0), (v1, i1), (v2, i2) = _merge3(
        ((v0, i0), (v1, i1), (v2, i2)), tuple(virt))

    i0_ref[...] = i0
    i1_ref[...] = i1
    i2_ref[...] = i2


def _knn(xp, xT2, brow, bcol, lo_t, hi_t, n, feat):
    npad = xp.shape[0]
    nt = npad // T
    body = functools.partial(_knn_body, n=n, feat=feat)
    grid_spec = pltpu.PrefetchScalarGridSpec(
        num_scalar_prefetch=2,
        grid=(nt,),
        in_specs=[
            pl.BlockSpec(xp.shape, lambda i, *_: (0, 0)),
            pl.BlockSpec(xT2.shape, lambda i, *_: (0, 0)),
            pl.BlockSpec(brow.shape, lambda i, *_: (0, 0)),
            pl.BlockSpec(bcol.shape, lambda i, *_: (0, 0)),
        ],
        out_specs=[
            pl.BlockSpec((T, 1), lambda i, *_: (i, 0)),
            pl.BlockSpec((T, 1), lambda i, *_: (i, 0)),
            pl.BlockSpec((T, 1), lambda i, *_: (i, 0)),
        ],
    )
    out_shape = [jax.ShapeDtypeStruct((npad, 1), jnp.int32)] * 3
    return pl.pallas_call(
        body, grid_spec=grid_spec, out_shape=out_shape,
        compiler_params=pltpu.CompilerParams(
            dimension_semantics=("parallel",)),
    )(lo_t, hi_t, xp, xT2, brow, bcol)


def _sc_gather(table, idx, width, chunks):
    nc, ns = 2, 16
    nw = nc * ns
    ep = idx.shape[0]
    bpw = ep // nw
    cs = bpw // chunks
    mesh = plsc.VectorSubcoreMesh(core_axis_name="c", subcore_axis_name="s")

    @functools.partial(
        pl.kernel, mesh=mesh,
        out_type=jax.ShapeDtypeStruct((ep, width), jnp.float32),
        scratch_types=[
            pltpu.VMEM((bpw,), jnp.int32),
            pltpu.VMEM((cs, width), jnp.float32),
            pltpu.VMEM((cs, width), jnp.float32),
            pltpu.SemaphoreType.DMA,
            pltpu.SemaphoreType.DMA,
        ])
    def k(table_hbm, idx_hbm, out_hbm, idx_v, rows_a, rows_b, sem_a, sem_b):
        wid = jax.lax.axis_index("s") * nc + jax.lax.axis_index("c")
        base = wid * bpw
        pltpu.sync_copy(idx_hbm.at[pl.ds(base, bpw)], idx_v)
        bufs = [(rows_a, sem_a), (rows_b, sem_b)]
        copies = []
        for c in range(chunks):
            src = idx_v if chunks == 1 else idx_v.at[pl.ds(c * cs, cs)]
            rows_v, sem = bufs[c % 2]
            copies.append(pltpu.async_copy(table_hbm.at[src], rows_v, sem))
        for c in range(chunks):
            rows_v, _ = bufs[c % 2]
            copies[c].wait()
            pltpu.sync_copy(rows_v, out_hbm.at[pl.ds(base + c * cs, cs)])

    return k(table, idx)


def _ec_common(x_ref, g_ref, w_ref, b_ref, ga_ref, be_ref, msg_ref, xn_ref,
               e_s, s1_s, s2_s, *, ne, do, de, nb):
    p = pl.program_id(0)
    b = pl.program_id(1)
    rb = x_ref.shape[0]
    e_s[:, 0:de] = x_ref[:, 0:de]

    def h_of(t):
        e_s[:, de:2 * de] = g_ref[pl.Slice(t, rb, 3), 0:de]
        return _mmdot(e_s[...], w_ref[...]) + b_ref[...]

    @pl.when(p == 0)
    def _():
        @pl.when(b == 0)
        def _():
            s1_s[...] = jnp.zeros_like(s1_s)
            s2_s[...] = jnp.zeros_like(s2_s)
        for t in range(3):
            h = h_of(t)
            s1_s[...] += jnp.sum(h, axis=0, keepdims=True)
            s2_s[...] += jnp.sum(h * h, axis=0, keepdims=True)

    @pl.when(p == 1)
    def _():
        mu = s1_s[...] / ne
        var = s2_s[...] / ne - mu * mu
        sd = jnp.sqrt(var + EPS)
        ga = ga_ref[...]
        be = be_ref[...]
        msum = jnp.zeros((rb, do), jnp.float32)
        for t in range(3):
            h = h_of(t)
            m = jnp.maximum((h - mu) / sd * ga + be, 0.0)
            msg_ref[pl.Slice(t, rb, 3), :] = m
            msum = msum + m
        xn_ref[...] = msum / 3.0


def _ec_body(x_ref, g_ref, w_ref, b_ref, ga_ref, be_ref, msg_ref, xn_ref,
             x1p_ref, x1w_ref, e_s, s1_s, s2_s, *, ne, do, de, nb):
    _ec_common(x_ref, g_ref, w_ref, b_ref, ga_ref, be_ref, msg_ref, xn_ref,
               e_s, s1_s, s2_s, ne=ne, do=do, de=de, nb=nb)
    p = pl.program_id(0)

    @pl.when(p == 1)
    def _():
        xn = xn_ref[...]
        x1p_ref[...] = xn
        x1w_ref[:, 0:do] = xn
        x1w_ref[:, do:2 * do] = jnp.zeros_like(xn)


def _edge_conv(X, Gn, wT, b_, ga_, be_, de, npad_out):
    n = X.shape[0]
    do = wT.shape[1]
    rb = RB if n % RB == 0 else n
    nb = n // rb
    body = functools.partial(_ec_body, ne=3 * n, do=do, de=de, nb=nb)
    return pl.pallas_call(
        body,
        grid=(2, nb),
        in_specs=[
            pl.BlockSpec((rb, X.shape[1]), lambda p, b: (b, 0)),
            pl.BlockSpec((3 * rb, Gn.shape[1]), lambda p, b: (b, 0)),
            pl.BlockSpec(wT.shape, lambda p, b: (0, 0)),
            pl.BlockSpec(b_.shape, lambda p, b: (0, 0)),
            pl.BlockSpec(ga_.shape, lambda p, b: (0, 0)),
            pl.BlockSpec(be_.shape, lambda p, b: (0, 0)),
        ],
        out_specs=[
            pl.BlockSpec((3 * rb, do), lambda p, b: (b, 0)),
            pl.BlockSpec((rb, do), lambda p, b: (b, 0)),
            pl.BlockSpec((rb, do), lambda p, b: (b, 0)),
            pl.BlockSpec((rb, 2 * do), lambda p, b: (b, 0)),
        ],
        out_shape=[jax.ShapeDtypeStruct((3 * n, do), jnp.float32),
                   jax.ShapeDtypeStruct((n, do), jnp.float32),
                   jax.ShapeDtypeStruct((npad_out, do), jnp.float32),
                   jax.ShapeDtypeStruct((n, 2 * do), jnp.float32)],
        scratch_shapes=[pltpu.VMEM((rb, 2 * de), jnp.float32),
                        pltpu.VMEM((1, do), jnp.float32),
                        pltpu.VMEM((1, do), jnp.float32)],
    )(X, Gn, wT, b_, ga_, be_)


def _ec_pool_body(x_ref, g_ref, w_ref, b_ref, ga_ref, be_ref, brow_ref,
                  f1w_ref, f1b_ref, f2w_ref, f2b_ref,
                  msg_ref, xn_ref, out_ref,
                  e_s, s1_s, s2_s, ps_s, cnt_s, *, ne, do, de, nb):
    _ec_common(x_ref, g_ref, w_ref, b_ref, ga_ref, be_ref, msg_ref, xn_ref,
               e_s, s1_s, s2_s, ne=ne, do=do, de=de, nb=nb)
    p = pl.program_id(0)
    b = pl.program_id(1)

    @pl.when(p == 1)
    def _():
        @pl.when(b == 0)
        def _():
            ps_s[...] = jnp.zeros_like(ps_s)
            cnt_s[...] = jnp.zeros_like(cnt_s)
        x2 = xn_ref[...]
        gid = jax.lax.broadcasted_iota(jnp.int32, (G, x2.shape[0]), 0)
        bb_row = brow_ref[...].reshape(1, x2.shape[0])
        oh = jnp.where(gid == bb_row, 1.0, 0.0).astype(jnp.float32)
        ps_s[...] += _mmdot(oh, x2, prec=jax.lax.Precision.HIGHEST)
        cnt_s[...] += jnp.sum(oh, axis=1, keepdims=True)

        @pl.when(b == nb - 1)
        def _():
            pooled = ps_s[...] / jnp.maximum(cnt_s[...], 1.0)
            o1 = _mmdot(pooled, f1w_ref[...]) + f1b_ref[...]
            out_ref[...] = _mmdot(o1, f2w_ref[...]) + f2b_ref[...]


def _edge_conv_pool(X, Gn, wT, b_, ga_, be_, brow_n, f1w, f1b, f2w, f2b, de):
    n = X.shape[0]
    do = wT.shape[1]
    rb = RB if n % RB == 0 else n
    nb = n // rb
    brow_n = brow_n.reshape(nb, 1, rb)
    body = functools.partial(_ec_pool_body, ne=3 * n, do=do, de=de, nb=nb)
    return pl.pallas_call(
        body,
        grid=(2, nb),
        in_specs=[
            pl.BlockSpec((rb, X.shape[1]), lambda p, b: (b, 0)),
            pl.BlockSpec((3 * rb, Gn.shape[1]), lambda p, b: (b, 0)),
            pl.BlockSpec(wT.shape, lambda p, b: (0, 0)),
            pl.BlockSpec(b_.shape, lambda p, b: (0, 0)),
            pl.BlockSpec(ga_.shape, lambda p, b: (0, 0)),
            pl.BlockSpec(be_.shape, lambda p, b: (0, 0)),
            pl.BlockSpec((1, 1, rb), lambda p, b: (b, 0, 0)),
            pl.BlockSpec(f1w.shape, lambda p, b: (0, 0)),
            pl.BlockSpec(f1b.shape, lambda p, b: (0, 0)),
            pl.BlockSpec(f2w.shape, lambda p, b: (0, 0)),
            pl.BlockSpec(f2b.shape, lambda p, b: (0, 0)),
        ],
        out_specs=[
            pl.BlockSpec((3 * rb, do), lambda p, b: (b, 0)),
            pl.BlockSpec((rb, do), lambda p, b: (b, 0)),
            pl.BlockSpec((G, 1), lambda p, b: (0, 0)),
        ],
        out_shape=[jax.ShapeDtypeStruct((3 * n, do), jnp.float32),
                   jax.ShapeDtypeStruct((n, do), jnp.float32),
                   jax.ShapeDtypeStruct((G, 1), jnp.float32)],
        scratch_shapes=[pltpu.VMEM((rb, 2 * de), jnp.float32),
                        pltpu.VMEM((1, do), jnp.float32),
                        pltpu.VMEM((1, do), jnp.float32),
                        pltpu.VMEM((G, do), jnp.float32),
                        pltpu.VMEM((G, 1), jnp.float32)],
    )(X, Gn, wT, b_, ga_, be_, brow_n, f1w, f1b, f2w, f2b)


def kernel(x, batch, w0, b0, g0, be0, w1, b1, g1, be1,
           fc1_w, fc1_b, fc2_w, fc2_b):
    n, dfeat = x.shape
    npad = ((n + T - 1) // T) * T
    nt = npad // T
    e = n * K
    ep = ((e + 255) // 256) * 256

    batch32 = batch.astype(jnp.int32)
    bpad = jnp.concatenate([batch32, jnp.full((npad - n,), G, jnp.int32)])
    brow = bpad.reshape(1, npad)
    bcol = bpad.reshape(npad, 1)

    tile_first = jnp.arange(nt) * T
    tile_last = jnp.minimum(tile_first + T - 1, n - 1)
    g_first = batch32[tile_first][:, None]
    g_last = batch32[tile_last][:, None]
    col_lo = jnp.sum(batch32[None, :] < g_first, axis=1)
    col_hi = jnp.sum(batch32[None, :] <= g_last, axis=1)
    lo_t = (col_lo // T).astype(jnp.int32)
    hi_t = ((col_hi - 1) // T).astype(jnp.int32)

    zpad = jnp.zeros((npad - n, dfeat), jnp.float32)
    xp = jnp.concatenate([x, zpad], axis=0)
    posT = xp[:, :2].T

    tgt = jnp.asarray(np.repeat(np.arange(n, dtype=np.int32), K))
    idxpad_tail = jnp.asarray(np.zeros((ep - e,), np.int32))

    i00, i01, i02 = _knn(xp, 2.0 * posT, brow, bcol, lo_t, hi_t, n, 2)
    src0 = jnp.concatenate([i00, i01, i02], axis=1)[:n].reshape(-1)
    g0rows = _sc_gather(x, jnp.concatenate([src0, idxpad_tail]), 128, 2)
    msg0, x1, x1p, x1w = _edge_conv(x, g0rows, w0.T, b0.reshape(1, -1),
                                    g0.reshape(1, -1), be0.reshape(1, -1),
                                    128, npad)

    x1T2 = 2.0 * x1p.T
    i10, i11, i12 = _knn(x1p, x1T2, brow, bcol, lo_t, hi_t, n, 64)
    src1 = jnp.concatenate([i10, i11, i12], axis=1)[:n].reshape(-1)
    g1rows = _sc_gather(x1w, jnp.concatenate([src1, idxpad_tail]), 128, 2)
    msg1, x2, out = _edge_conv_pool(
        x1w, g1rows, w1.T, b1.reshape(1, -1), g1.reshape(1, -1),
        be1.reshape(1, -1), brow[:, :n], fc1_w.T, fc1_b.reshape(1, -1),
        fc2_w.T, fc2_b.reshape(1, -1), 64)

    ea0 = msg0
    ea1 = msg1
    ei0 = jnp.stack([src0, tgt], axis=0).astype(jnp.int64)
    ei1 = jnp.stack([src1, tgt], axis=0).astype(jnp.int64)
    return (out, (ea0, ea1), (x1, x2), (ei0, ei1))
